# baseline (speedup 1.0000x reference)
import jax
import jax.numpy as jnp
from jax import lax
from jax.experimental import pallas as pl
from jax.experimental.pallas import tpu as pltpu

B, Q, L = 64, 100, 256
BB = 32
N = 128


def _postproc_kernel(logits_ref, boxes_ref, ts_ref, posmap_ref, out_ref):
    x = logits_ref[...]
    m = jnp.max(x, axis=-1, keepdims=True)
    e = jnp.exp(x - m)
    s = jnp.sum(e, axis=-1, keepdims=True)
    pos = posmap_ref[...][:, None, :] > 1e-6
    emax = jnp.max(jnp.where(pos, e, 0.0), axis=-1, keepdims=True)
    score = emax / s

    ts = ts_ref[...].astype(jnp.float32)
    img_h = ts[:, 0:1][:, None, :]
    img_w = ts[:, 1:2][:, None, :]

    bx = boxes_ref[...]
    cx = bx[:, :, 0:1]
    cy = bx[:, :, 1:2]
    w = bx[:, :, 2:3]
    h = bx[:, :, 3:4]
    xyxy = jnp.concatenate(
        [
            (cx - 0.5 * w) * img_w,
            (cy - 0.5 * h) * img_h,
            (cx + 0.5 * w) * img_w,
            (cy + 0.5 * h) * img_h,
        ],
        axis=-1,
    )

    score_row = jnp.swapaxes(score, 1, 2)
    key = jnp.reshape(
        jnp.concatenate(
            [score_row, jnp.full((BB, 1, N - Q), -1.0, jnp.float32)], axis=2
        ),
        (BB, N),
    )
    idx = jnp.broadcast_to(lax.broadcasted_iota(jnp.int32, (1, N), 1), (BB, N))
    lane = lax.broadcasted_iota(jnp.int32, (1, N), 1)

    k = 2
    while k <= N:
        j = k // 2
        while j >= 1:
            low = (lane & j) == 0
            pkey = jnp.where(low, pltpu.roll(key, N - j, 1), pltpu.roll(key, j, 1))
            pidx = jnp.where(low, pltpu.roll(idx, N - j, 1), pltpu.roll(idx, j, 1))
            before = (key > pkey) | ((key == pkey) & (idx < pidx))
            dsc = (lane & k) == 0
            keep = (before == low) == dsc
            key = jnp.where(keep, key, pkey)
            idx = jnp.where(keep, idx, pidx)
            j //= 2
        k *= 2

    sidx = idx[:, None, :Q]
    ii = lax.broadcasted_iota(jnp.int32, (1, Q, 1), 1)
    oneT = (ii == sidx).astype(jnp.float32)

    out_ref[...] = lax.dot_general(
        oneT,
        xyxy,
        dimension_numbers=(((1,), (1,)), ((0,), (0,))),
        preferred_element_type=jnp.float32,
        precision=lax.Precision.HIGHEST,
    )


def kernel(pred_logits, pred_boxes, target_sizes, positive_map, items_per_batch_element):
    del items_per_batch_element
    return pl.pallas_call(
        _postproc_kernel,
        grid=(B // BB,),
        in_specs=[
            pl.BlockSpec((BB, Q, L), lambda b: (b, 0, 0)),
            pl.BlockSpec((BB, Q, 4), lambda b: (b, 0, 0)),
            pl.BlockSpec((BB, 2), lambda b: (b, 0)),
            pl.BlockSpec((BB, L), lambda b: (b, 0)),
        ],
        out_specs=pl.BlockSpec((BB, Q, 4), lambda b: (b, 0, 0)),
        out_shape=jax.ShapeDtypeStruct((B, Q, 4), jnp.float32),
        compiler_params=pltpu.CompilerParams(
            dimension_semantics=("parallel",),
        ),
    )(pred_logits, pred_boxes, target_sizes, positive_map)


# speedup vs baseline: 1.1441x; 1.1441x over previous
import jax
import jax.numpy as jnp
from jax import lax
from jax.experimental import pallas as pl
from jax.experimental.pallas import tpu as pltpu

B, Q, L = 64, 100, 256
BB = 32


def _postproc_kernel(logits_ref, boxes_ref, ts_ref, posmap_ref, out_ref):
    x = logits_ref[...]
    m = jnp.max(x, axis=-1, keepdims=True)
    e = jnp.exp(x - m)
    s = jnp.sum(e, axis=-1, keepdims=True)
    pos = posmap_ref[...][:, None, :] > 1e-6
    emax = jnp.max(jnp.where(pos, e, 0.0), axis=-1, keepdims=True)
    score = emax / s

    ts = ts_ref[...].astype(jnp.float32)
    img_h = ts[:, 0:1][:, None, :]
    img_w = ts[:, 1:2][:, None, :]

    bx = boxes_ref[...]
    rolled = jnp.concatenate([bx[:, :, 2:4], bx[:, :, 0:2]], axis=-1)
    lane = lax.broadcasted_iota(jnp.int32, (1, 1, 4), 2)
    first2 = lane < 2
    cxcy2 = jnp.where(first2, bx, rolled)
    whwh = jnp.where(first2, rolled, bx)
    sgn = jnp.where(first2, -0.5, 0.5)
    axscale = jnp.where(lane % 2 == 0, img_w, img_h)
    xyxy = (cxcy2 + sgn * whwh) * axscale

    score_row = jnp.swapaxes(score, 1, 2)
    ii = lax.broadcasted_iota(jnp.int32, (1, Q, Q), 1)
    jj = lax.broadcasted_iota(jnp.int32, (1, Q, Q), 2)

    beats = (score_row > score) | ((score_row == score) & (jj < ii))
    rank = lax.dot_general(
        beats.astype(jnp.float32),
        jnp.ones((Q, 1), jnp.float32),
        dimension_numbers=(((2,), (0,)), ((), ())),
        preferred_element_type=jnp.float32,
        precision=lax.Precision.HIGHEST,
    )

    rr = lax.broadcasted_iota(jnp.int32, (1, 1, Q), 2).astype(jnp.float32)
    takeT = (rank == rr).astype(jnp.float32)

    out_ref[...] = lax.dot_general(
        takeT,
        xyxy,
        dimension_numbers=(((1,), (1,)), ((0,), (0,))),
        preferred_element_type=jnp.float32,
        precision=lax.Precision.HIGHEST,
    )


def kernel(pred_logits, pred_boxes, target_sizes, positive_map, items_per_batch_element):
    del items_per_batch_element
    return pl.pallas_call(
        _postproc_kernel,
        grid=(B // BB,),
        in_specs=[
            pl.BlockSpec((BB, Q, L), lambda b: (b, 0, 0)),
            pl.BlockSpec((BB, Q, 4), lambda b: (b, 0, 0)),
            pl.BlockSpec((BB, 2), lambda b: (b, 0)),
            pl.BlockSpec((BB, L), lambda b: (b, 0)),
        ],
        out_specs=pl.BlockSpec((BB, Q, 4), lambda b: (b, 0, 0)),
        out_shape=jax.ShapeDtypeStruct((B, Q, 4), jnp.float32),
        compiler_params=pltpu.CompilerParams(
            dimension_semantics=("parallel",),
        ),
    )(pred_logits, pred_boxes, target_sizes, positive_map)


# speedup vs baseline: 1.2786x; 1.1176x over previous
import jax
import jax.numpy as jnp
from jax import lax
from jax.experimental import pallas as pl
from jax.experimental.pallas import tpu as pltpu

B, Q, L = 64, 100, 256
BB = 32


def _postproc_kernel(logits_ref, boxes_ref, ts_ref, posmap_ref, out_ref):
    x = logits_ref[...]
    m = jnp.max(x, axis=-1, keepdims=True)
    s = jnp.sum(jnp.exp(x - m), axis=-1, keepdims=True)
    pos = posmap_ref[...][:, None, :] > 1e-6
    mm = jnp.max(jnp.where(pos, x, -jnp.inf), axis=-1, keepdims=True)
    score = jnp.exp(mm - m) / s

    ts = ts_ref[...].astype(jnp.float32)
    img_h = ts[:, 0:1][:, None, :]
    img_w = ts[:, 1:2][:, None, :]

    bx = boxes_ref[...]
    cx = bx[:, :, 0:1]
    cy = bx[:, :, 1:2]
    w = bx[:, :, 2:3]
    h = bx[:, :, 3:4]
    xyxy = jnp.concatenate(
        [
            (cx - 0.5 * w) * img_w,
            (cy - 0.5 * h) * img_h,
            (cx + 0.5 * w) * img_w,
            (cy + 0.5 * h) * img_h,
        ],
        axis=-1,
    )

    score_row = jnp.swapaxes(score, 1, 2)
    ii = lax.broadcasted_iota(jnp.int32, (1, Q, Q), 1)
    jj = lax.broadcasted_iota(jnp.int32, (1, Q, Q), 2)

    beats = (score_row > score) | ((score_row == score) & (jj < ii))
    rank = jnp.sum(beats.astype(jnp.int32), axis=2, keepdims=True)

    rr = lax.broadcasted_iota(jnp.int32, (1, 1, Q), 2)
    takeT = (rank == rr).astype(jnp.float32)

    out_ref[...] = lax.dot_general(
        takeT,
        xyxy,
        dimension_numbers=(((1,), (1,)), ((0,), (0,))),
        preferred_element_type=jnp.float32,
        precision=lax.Precision.HIGHEST,
    )


def kernel(pred_logits, pred_boxes, target_sizes, positive_map, items_per_batch_element):
    del items_per_batch_element
    return pl.pallas_call(
        _postproc_kernel,
        grid=(B // BB,),
        in_specs=[
            pl.BlockSpec((BB, Q, L), lambda b: (b, 0, 0)),
            pl.BlockSpec((BB, Q, 4), lambda b: (b, 0, 0)),
            pl.BlockSpec((BB, 2), lambda b: (b, 0)),
            pl.BlockSpec((BB, L), lambda b: (b, 0)),
        ],
        out_specs=pl.BlockSpec((BB, Q, 4), lambda b: (b, 0, 0)),
        out_shape=jax.ShapeDtypeStruct((B, Q, 4), jnp.float32),
        compiler_params=pltpu.CompilerParams(
            dimension_semantics=("parallel",),
        ),
    )(pred_logits, pred_boxes, target_sizes, positive_map)
